# baseline (speedup 1.0000x reference)
import jax
import jax.numpy as jnp
from jax import lax
from jax.experimental import pallas as pl
from jax.experimental.pallas import tpu as pltpu


_BLOCK = 131072
_CHUNK = 8192


def _ce_kernel(x_ref, y_ref, out_ref):
    j = pl.program_id(1)
    C = x_ref.shape[1]
    ones = jnp.ones((1, C), jnp.float32)
    dn = (((1,), (0,)), ((), ()))

    def body(k, acc):
        sl = pl.ds(k * _CHUNK, _CHUNK)
        xt = x_ref[0, :, sl]
        yt = y_ref[0, :, sl]

        e = jnp.exp(xt)
        cls = lax.broadcasted_iota(jnp.int32, xt.shape, 0)
        masked = jnp.where(cls == yt, xt, 0.0)

        s = lax.dot_general(ones, e, dn,
                            preferred_element_type=jnp.float32)
        x_true = lax.dot_general(ones, masked, dn,
                                 preferred_element_type=jnp.float32)
        return acc + jnp.sum(jnp.log(s) - x_true)

    acc = lax.fori_loop(0, _BLOCK // _CHUNK, body, jnp.float32(0.0))
    acc = acc.reshape(1, 1, 1)

    @pl.when(j == 0)
    def _first():
        out_ref[...] = acc

    @pl.when(j > 0)
    def _rest():
        out_ref[...] += acc


def kernel(x, y):
    B, C = x.shape[0], x.shape[1]
    HW = x.shape[2] * x.shape[3]
    x = x.reshape(B, C, HW)
    y = y.reshape(B, 1, HW).astype(jnp.int32)

    partial = pl.pallas_call(
        _ce_kernel,
        grid=(B, HW // _BLOCK),
        in_specs=[
            pl.BlockSpec((1, C, _BLOCK), lambda b, j: (b, 0, j)),
            pl.BlockSpec((1, 1, _BLOCK), lambda b, j: (b, 0, j)),
        ],
        out_specs=pl.BlockSpec((1, 1, 1), lambda b, j: (b, 0, 0)),
        out_shape=jax.ShapeDtypeStruct((B, 1, 1), jnp.float32),
        compiler_params=pltpu.CompilerParams(
            dimension_semantics=("parallel", "arbitrary"),
            vmem_limit_bytes=100 * 1024 * 1024,
        ),
    )(x, y)

    return jnp.sum(partial) / jnp.float32(B * HW)


# speedup vs baseline: 1.1047x; 1.1047x over previous
import jax
import jax.numpy as jnp
from jax.experimental import pallas as pl
from jax.experimental.pallas import tpu as pltpu


_CHUNK = 65536


def _ce_kernel(x_ref, y_ref, out_ref):
    j = pl.program_id(1)

    xt = x_ref[0]
    yt = y_ref[0]

    e = jnp.exp(xt)
    cls = jax.lax.broadcasted_iota(jnp.int32, xt.shape, 0)
    masked = jnp.where(cls == yt, xt, 0.0)

    ones = jnp.ones((1, xt.shape[0]), jnp.float32)
    dn = (((1,), (0,)), ((), ()))
    s = jax.lax.dot_general(ones, e, dn,
                            preferred_element_type=jnp.float32)
    x_true = jax.lax.dot_general(ones, masked, dn,
                                 preferred_element_type=jnp.float32)

    tile_sum = jnp.sum(jnp.log(s) - x_true).reshape(1, 1, 1)

    @pl.when(j == 0)
    def _init():
        out_ref[...] = jnp.zeros((1, 1, 1), jnp.float32)

    out_ref[...] += tile_sum


def kernel(x, y):
    B, C = x.shape[0], x.shape[1]
    HW = x.shape[2] * x.shape[3]
    x = x.reshape(B, C, HW)
    y = y.reshape(B, 1, HW).astype(jnp.int32)

    n_chunks = HW // _CHUNK

    partial = pl.pallas_call(
        _ce_kernel,
        grid=(B, n_chunks),
        in_specs=[
            pl.BlockSpec((1, C, _CHUNK), lambda b, j: (b, 0, j)),
            pl.BlockSpec((1, 1, _CHUNK), lambda b, j: (b, 0, j)),
        ],
        out_specs=pl.BlockSpec((1, 1, 1), lambda b, j: (b, 0, 0)),
        out_shape=jax.ShapeDtypeStruct((B, 1, 1), jnp.float32),
        compiler_params=pltpu.CompilerParams(
            dimension_semantics=("parallel", "arbitrary"),
        ),
    )(x, y)

    return jnp.sum(partial) / jnp.float32(B * HW)
